# Initial kernel scaffold; baseline (speedup 1.0000x reference)
import jax
import jax.numpy as jnp
from jax.experimental import pallas as pl


def kernel(x, W1, b1, Wa1, ba1, Wa2, ba2, Wc, bc, rehearsal, Wg, bg, Wgc1, bgc1, Wgc2, bgc2):
    raise NotImplementedError("write your pallas kernel here")


# speedup vs baseline: 31.3567x; 31.3567x over previous
import functools

import jax
import jax.numpy as jnp
from jax import lax
from jax.experimental import pallas as pl
from jax.experimental.pallas import tpu as pltpu
from jax.experimental.pallas import tpu_sc as plsc

B_SZ = 16
N_INST = 1024
F_IN = 1024
LP = 512
K = 16
N_NODES = 4112
NC, NS = 2, 16
NW = NC * NS


def _pad_cols(a, cols):
    return jnp.pad(a, ((0, 0), (0, cols - a.shape[1])))


def _pad_rows(a, rows):
    return jnp.pad(a, ((0, rows - a.shape[0]), (0, 0)))


def _abmil_body(x_ref, w1_ref, b1_ref, wa1_ref, ba1_ref, wa2t_ref, ba2_ref,
                wc_ref, bc_ref, slide_ref, logits_ref):
    x = x_ref[0]
    h = jnp.maximum(
        jnp.dot(x, w1_ref[...], preferred_element_type=jnp.float32)
        + b1_ref[...], 0.0)
    a = jnp.maximum(
        jnp.dot(h, wa1_ref[...], preferred_element_type=jnp.float32)
        + ba1_ref[...], 0.0)
    s = jnp.sum(a * wa2t_ref[...], axis=1, keepdims=True) + ba2_ref[...]
    m = jnp.max(s, axis=0, keepdims=True)
    e = jnp.exp(s - m)
    w = e / jnp.sum(e, axis=0, keepdims=True)
    slide = jnp.sum(h * w, axis=0, keepdims=True)
    slide_ref[0] = slide
    logits_ref[0] = (
        jnp.dot(slide, wc_ref[...], preferred_element_type=jnp.float32)
        + bc_ref[...])


def _abmil(x, w1p, b1p, wa1p, ba1, wa2t, ba2, wcp, bc):
    return pl.pallas_call(
        _abmil_body,
        grid=(B_SZ,),
        in_specs=[
            pl.BlockSpec((1, N_INST, F_IN), lambda b: (b, 0, 0)),
            pl.BlockSpec((F_IN, LP), lambda b: (0, 0)),
            pl.BlockSpec((1, LP), lambda b: (0, 0)),
            pl.BlockSpec((LP, 128), lambda b: (0, 0)),
            pl.BlockSpec((1, 128), lambda b: (0, 0)),
            pl.BlockSpec((1, 128), lambda b: (0, 0)),
            pl.BlockSpec((1, 1), lambda b: (0, 0)),
            pl.BlockSpec((LP, 2), lambda b: (0, 0)),
            pl.BlockSpec((1, 2), lambda b: (0, 0)),
        ],
        out_specs=[
            pl.BlockSpec((1, 1, LP), lambda b: (b, 0, 0)),
            pl.BlockSpec((1, 1, 2), lambda b: (b, 0, 0)),
        ],
        out_shape=[
            jax.ShapeDtypeStruct((B_SZ, 1, LP), jnp.float32),
            jax.ShapeDtypeStruct((B_SZ, 1, 2), jnp.float32),
        ],
    )(x, w1p, b1p, wa1p, ba1, wa2t, ba2, wcp, bc)


def _embed_body(xc_ref, wg_ref, bg_ref, en_ref):
    e = jnp.tanh(
        jnp.dot(xc_ref[...], wg_ref[...], preferred_element_type=jnp.float32)
        + bg_ref[...])
    nrm = jnp.sqrt(jnp.sum(e * e, axis=1, keepdims=True))
    en_ref[...] = e / (nrm + 1e-8)


def _embed(xcp, wgp, bgp):
    return pl.pallas_call(
        _embed_body,
        out_shape=jax.ShapeDtypeStruct((N_NODES, LP), jnp.float32),
    )(xcp, wgp, bgp)


def _topk_body(rows_ref, en_ref, idx_ref, attr_ref):
    rows = rows_ref[...]
    sim = lax.dot_general(rows, en_ref[...],
                          (((1,), (1,)), ((), ())),
                          preferred_element_type=jnp.float32)
    r, n = sim.shape
    col = lax.broadcasted_iota(jnp.int32, (r, n), 1)
    vals, idxs = [], []
    cur = sim
    for _ in range(K):
        m = jnp.max(cur, axis=1, keepdims=True)
        am = jnp.min(jnp.where(cur == m, col, n), axis=1, keepdims=True)
        vals.append(m)
        idxs.append(am)
        cur = jnp.where(col == am, -jnp.inf, cur)
    v = jnp.concatenate(vals, axis=1)
    idx_ref[...] = jnp.concatenate(idxs, axis=1)
    ev = jnp.exp(v - v[:, 0:1])
    attr_ref[...] = ev / jnp.sum(ev, axis=1, keepdims=True)


def _topk(rows, en):
    r = rows.shape[0]
    return pl.pallas_call(
        _topk_body,
        out_shape=[
            jax.ShapeDtypeStruct((r, K), jnp.int32),
            jax.ShapeDtypeStruct((r, K), jnp.float32),
        ],
    )(rows, en)


def _sc_gather_rows(table, idx):
    b = idx.shape[0]
    d = table.shape[1]
    b_per_w = b // NW
    mesh = plsc.VectorSubcoreMesh(core_axis_name="c", subcore_axis_name="s")

    @functools.partial(
        pl.kernel,
        mesh=mesh,
        out_type=jax.ShapeDtypeStruct((b, d), jnp.float32),
        scratch_types=[
            pltpu.VMEM((b_per_w,), jnp.int32),
            pltpu.VMEM((b_per_w, d), jnp.float32),
            pltpu.SemaphoreType.DMA,
        ],
    )
    def gather_kernel(table_hbm, idx_hbm, out_hbm, idx_v, rows_v, sem):
        wid = lax.axis_index("s") * NC + lax.axis_index("c")
        base = wid * b_per_w
        pltpu.sync_copy(idx_hbm.at[pl.ds(base, b_per_w)], idx_v)
        pltpu.async_copy(table_hbm.at[idx_v], rows_v, sem).wait()
        pltpu.sync_copy(rows_v, out_hbm.at[pl.ds(base, b_per_w)])

    return gather_kernel(table, idx)


def _gcn_body(gx_ref, attrj_ref, wgc1_ref, bgc1_ref, attr0_ref, wgc2_ref,
              bgc2_ref, out_ref):
    ne, d = gx_ref.shape
    nj = ne // K
    row = lax.broadcasted_iota(jnp.int32, (nj, ne), 0)
    col = lax.broadcasted_iota(jnp.int32, (nj, ne), 1)
    wmat = jnp.where(col // K == row, attrj_ref[...], 0.0)
    agg1 = jnp.dot(wmat, gx_ref[...],
                   preferred_element_type=jnp.float32)
    h1 = jnp.maximum(
        jnp.dot(agg1, wgc1_ref[...], preferred_element_type=jnp.float32)
        + bgc1_ref[...], 0.0)
    row2 = lax.broadcasted_iota(jnp.int32, (B_SZ, nj), 0)
    col2 = lax.broadcasted_iota(jnp.int32, (B_SZ, nj), 1)
    wmat2 = jnp.where(col2 // K == row2, attr0_ref[...], 0.0)
    agg2 = jnp.dot(wmat2, h1, preferred_element_type=jnp.float32)
    out_ref[...] = (
        jnp.dot(agg2, wgc2_ref[...], preferred_element_type=jnp.float32)
        + bgc2_ref[...])


def _gcn(gx, attrj_flat, wgc1p, bgc1p, attr0_flat, wgc2p, bgc2):
    return pl.pallas_call(
        _gcn_body,
        out_shape=jax.ShapeDtypeStruct((B_SZ, 2), jnp.float32),
    )(gx, attrj_flat, wgc1p, bgc1p, attr0_flat, wgc2p, bgc2)


def kernel(x, W1, b1, Wa1, ba1, Wa2, ba2, Wc, bc, rehearsal,
           Wg, bg, Wgc1, bgc1, Wgc2, bgc2):
    f32 = jnp.float32
    w1p = _pad_cols(W1, LP)
    b1p = _pad_cols(b1[None, :], LP)
    wa1p = _pad_rows(Wa1, LP)
    wcp = _pad_rows(Wc, LP)
    wgp = _pad_cols(_pad_rows(Wg, LP), LP)
    bgp = _pad_cols(bg[None, :], LP)
    wgc1p = _pad_cols(_pad_rows(Wgc1, LP), LP)
    bgc1p = _pad_cols(bgc1[None, :], LP)
    wgc2p = _pad_rows(Wgc2, LP)

    slide_p, logits_mlp = _abmil(
        x, w1p, b1p, wa1p, ba1[None, :].astype(f32), Wa2.T.astype(f32),
        ba2[None, :].astype(f32), wcp, bc[None, :].astype(f32))
    slide_p = slide_p[:, 0, :]
    logits_mlp = logits_mlp[:, 0, :]

    xcp = jnp.concatenate([slide_p, _pad_cols(rehearsal, LP)], axis=0)
    en = _embed(xcp, wgp, bgp)

    idx0, attr0 = _topk(en[:B_SZ], en)
    j_idx = idx0.reshape(B_SZ * K)
    en_j = _sc_gather_rows(en, j_idx)

    idxj, attrj = _topk(en_j, en)
    src = idxj.reshape(B_SZ * K * K)
    gx = _sc_gather_rows(xcp, src)

    logits_graph = _gcn(gx, attrj.reshape(1, B_SZ * K * K), wgc1p, bgc1p,
                        attr0.reshape(1, B_SZ * K), wgc2p, bgc2[None, :])
    return (logits_mlp, logits_graph)


# speedup vs baseline: 33.3213x; 1.0627x over previous
import functools

import jax
import jax.numpy as jnp
from jax import lax
from jax.experimental import pallas as pl
from jax.experimental.pallas import tpu as pltpu
from jax.experimental.pallas import tpu_sc as plsc

B_SZ = 16
N_INST = 1024
F_IN = 1024
LP = 512
K = 16
N_NODES = 4112
NC, NS = 2, 16
NW = NC * NS


def _pad_cols(a, cols):
    return jnp.pad(a, ((0, 0), (0, cols - a.shape[1])))


def _pad_rows(a, rows):
    return jnp.pad(a, ((0, rows - a.shape[0]), (0, 0)))


def _abmil_body(x_ref, w1_ref, b1_ref, wa1_ref, ba1_ref, wa2t_ref, ba2_ref,
                wc_ref, bc_ref, slide_ref, logits_ref):
    x = x_ref[0]
    h = jnp.maximum(
        jnp.dot(x, w1_ref[...], preferred_element_type=jnp.float32)
        + b1_ref[...], 0.0)
    a = jnp.maximum(
        jnp.dot(h, wa1_ref[...], preferred_element_type=jnp.float32)
        + ba1_ref[...], 0.0)
    s = jnp.sum(a * wa2t_ref[...], axis=1, keepdims=True) + ba2_ref[...]
    m = jnp.max(s, axis=0, keepdims=True)
    e = jnp.exp(s - m)
    w = e / jnp.sum(e, axis=0, keepdims=True)
    slide = jnp.sum(h * w, axis=0, keepdims=True)
    slide_ref[0] = slide
    logits_ref[0] = (
        jnp.dot(slide, wc_ref[...], preferred_element_type=jnp.float32)
        + bc_ref[...])


def _abmil(x, w1p, b1p, wa1p, ba1, wa2t, ba2, wcp, bc):
    return pl.pallas_call(
        _abmil_body,
        grid=(B_SZ,),
        in_specs=[
            pl.BlockSpec((1, N_INST, F_IN), lambda b: (b, 0, 0)),
            pl.BlockSpec((F_IN, LP), lambda b: (0, 0)),
            pl.BlockSpec((1, LP), lambda b: (0, 0)),
            pl.BlockSpec((LP, 128), lambda b: (0, 0)),
            pl.BlockSpec((1, 128), lambda b: (0, 0)),
            pl.BlockSpec((1, 128), lambda b: (0, 0)),
            pl.BlockSpec((1, 1), lambda b: (0, 0)),
            pl.BlockSpec((LP, 2), lambda b: (0, 0)),
            pl.BlockSpec((1, 2), lambda b: (0, 0)),
        ],
        out_specs=[
            pl.BlockSpec((1, 1, LP), lambda b: (b, 0, 0)),
            pl.BlockSpec((1, 1, 2), lambda b: (b, 0, 0)),
        ],
        out_shape=[
            jax.ShapeDtypeStruct((B_SZ, 1, LP), jnp.float32),
            jax.ShapeDtypeStruct((B_SZ, 1, 2), jnp.float32),
        ],
    )(x, w1p, b1p, wa1p, ba1, wa2t, ba2, wcp, bc)


def _embed_body(slide_ref, reh_ref, wg_ref, bg_ref,
                en_ref, xcp_ref, idx_ref, attr_ref):
    s = slide_ref[...]
    r = reh_ref[...]
    e1 = jnp.tanh(
        jnp.dot(s, wg_ref[...], preferred_element_type=jnp.float32)
        + bg_ref[...])
    en1 = e1 / (jnp.sqrt(jnp.sum(e1 * e1, axis=1, keepdims=True)) + 1e-8)
    e2 = jnp.tanh(
        jnp.dot(r, wg_ref[0:500, :], preferred_element_type=jnp.float32)
        + bg_ref[...])
    en2 = e2 / (jnp.sqrt(jnp.sum(e2 * e2, axis=1, keepdims=True)) + 1e-8)
    en_ref[0:B_SZ] = en1
    en_ref[B_SZ:] = en2
    xcp_ref[0:B_SZ] = s
    xcp_ref[B_SZ:] = jnp.concatenate(
        [r, jnp.zeros((r.shape[0], LP - r.shape[1]), jnp.float32)], axis=1)
    sim = jnp.concatenate(
        [lax.dot_general(en1, en1, (((1,), (1,)), ((), ())),
                         preferred_element_type=jnp.float32),
         lax.dot_general(en1, en2, (((1,), (1,)), ((), ())),
                         preferred_element_type=jnp.float32)], axis=1)
    n = sim.shape[1]
    col = lax.broadcasted_iota(jnp.int32, (B_SZ, n), 1)
    vals, idxs = [], []
    cur = sim
    for _ in range(K):
        m = jnp.max(cur, axis=1, keepdims=True)
        am = jnp.min(jnp.where(cur == m, col, n), axis=1, keepdims=True)
        vals.append(m)
        idxs.append(am)
        cur = jnp.where(col == am, -jnp.inf, cur)
    v = jnp.concatenate(vals, axis=1)
    idx_ref[...] = jnp.concatenate(idxs, axis=1)
    ev = jnp.exp(v - v[:, 0:1])
    attr_ref[...] = ev / jnp.sum(ev, axis=1, keepdims=True)


def _embed(slide_p, rehearsal, wgp, bgp):
    return pl.pallas_call(
        _embed_body,
        out_shape=[
            jax.ShapeDtypeStruct((N_NODES, LP), jnp.float32),
            jax.ShapeDtypeStruct((N_NODES, LP), jnp.float32),
            jax.ShapeDtypeStruct((B_SZ, K), jnp.int32),
            jax.ShapeDtypeStruct((B_SZ, K), jnp.float32),
        ],
    )(slide_p, rehearsal, wgp, bgp)


def _topk_body(rows_ref, en_ref, idx_ref, attr_ref):
    rows = rows_ref[...]
    sim = lax.dot_general(rows, en_ref[...],
                          (((1,), (1,)), ((), ())),
                          preferred_element_type=jnp.float32)
    r, n = sim.shape
    col = lax.broadcasted_iota(jnp.int32, (r, n), 1)
    vals, idxs = [], []
    cur = sim
    for _ in range(K):
        m = jnp.max(cur, axis=1, keepdims=True)
        am = jnp.min(jnp.where(cur == m, col, n), axis=1, keepdims=True)
        vals.append(m)
        idxs.append(am)
        cur = jnp.where(col == am, -jnp.inf, cur)
    v = jnp.concatenate(vals, axis=1)
    idx_ref[...] = jnp.concatenate(idxs, axis=1)
    ev = jnp.exp(v - v[:, 0:1])
    attr_ref[...] = ev / jnp.sum(ev, axis=1, keepdims=True)


def _topk(rows, en):
    r = rows.shape[0]
    return pl.pallas_call(
        _topk_body,
        out_shape=[
            jax.ShapeDtypeStruct((r, K), jnp.int32),
            jax.ShapeDtypeStruct((r, K), jnp.float32),
        ],
    )(rows, en)


def _sc_gather_rows(table, idx):
    b = idx.shape[0]
    d = table.shape[1]
    b_per_w = b // NW
    mesh = plsc.VectorSubcoreMesh(core_axis_name="c", subcore_axis_name="s")

    @functools.partial(
        pl.kernel,
        mesh=mesh,
        out_type=jax.ShapeDtypeStruct((b, d), jnp.float32),
        scratch_types=[
            pltpu.VMEM((b_per_w,), jnp.int32),
            pltpu.VMEM((b_per_w, d), jnp.float32),
            pltpu.SemaphoreType.DMA,
        ],
    )
    def gather_kernel(table_hbm, idx_hbm, out_hbm, idx_v, rows_v, sem):
        wid = lax.axis_index("s") * NC + lax.axis_index("c")
        base = wid * b_per_w
        pltpu.sync_copy(idx_hbm.at[pl.ds(base, b_per_w)], idx_v)
        pltpu.async_copy(table_hbm.at[idx_v], rows_v, sem).wait()
        pltpu.sync_copy(rows_v, out_hbm.at[pl.ds(base, b_per_w)])

    return gather_kernel(table, idx)


def _gcn_body(gx_ref, attrj_ref, wgc1_ref, bgc1_ref, attr0_ref, wgc2_ref,
              bgc2_ref, out_ref):
    ne, d = gx_ref.shape
    nj = ne // K
    row = lax.broadcasted_iota(jnp.int32, (nj, ne), 0)
    col = lax.broadcasted_iota(jnp.int32, (nj, ne), 1)
    wmat = jnp.where(col // K == row, attrj_ref[...], 0.0)
    agg1 = jnp.dot(wmat, gx_ref[...],
                   preferred_element_type=jnp.float32)
    h1 = jnp.maximum(
        jnp.dot(agg1, wgc1_ref[...], preferred_element_type=jnp.float32)
        + bgc1_ref[...], 0.0)
    row2 = lax.broadcasted_iota(jnp.int32, (B_SZ, nj), 0)
    col2 = lax.broadcasted_iota(jnp.int32, (B_SZ, nj), 1)
    wmat2 = jnp.where(col2 // K == row2, attr0_ref[...], 0.0)
    agg2 = jnp.dot(wmat2, h1, preferred_element_type=jnp.float32)
    out_ref[...] = (
        jnp.dot(agg2, wgc2_ref[...], preferred_element_type=jnp.float32)
        + bgc2_ref[...])


def _gcn(gx, attrj_flat, wgc1p, bgc1p, attr0_flat, wgc2p, bgc2):
    return pl.pallas_call(
        _gcn_body,
        out_shape=jax.ShapeDtypeStruct((B_SZ, 2), jnp.float32),
    )(gx, attrj_flat, wgc1p, bgc1p, attr0_flat, wgc2p, bgc2)


def kernel(x, W1, b1, Wa1, ba1, Wa2, ba2, Wc, bc, rehearsal,
           Wg, bg, Wgc1, bgc1, Wgc2, bgc2):
    f32 = jnp.float32
    w1p = _pad_cols(W1, LP)
    b1p = _pad_cols(b1[None, :], LP)
    wa1p = _pad_rows(Wa1, LP)
    wcp = _pad_rows(Wc, LP)
    wgp = _pad_cols(_pad_rows(Wg, LP), LP)
    bgp = _pad_cols(bg[None, :], LP)
    wgc1p = _pad_cols(_pad_rows(Wgc1, LP), LP)
    bgc1p = _pad_cols(bgc1[None, :], LP)
    wgc2p = _pad_rows(Wgc2, LP)

    slide_p, logits_mlp = _abmil(
        x, w1p, b1p, wa1p, ba1[None, :].astype(f32), Wa2.T.astype(f32),
        ba2[None, :].astype(f32), wcp, bc[None, :].astype(f32))
    slide_p = slide_p[:, 0, :]
    logits_mlp = logits_mlp[:, 0, :]

    en, xcp, idx0, attr0 = _embed(slide_p, rehearsal, wgp, bgp)
    j_idx = idx0.reshape(B_SZ * K)
    en_j = _sc_gather_rows(en, j_idx)

    idxj, attrj = _topk(en_j, en)
    src = idxj.reshape(B_SZ * K * K)
    gx = _sc_gather_rows(xcp, src)

    logits_graph = _gcn(gx, attrj.reshape(1, B_SZ * K * K), wgc1p, bgc1p,
                        attr0.reshape(1, B_SZ * K), wgc2p, bgc2[None, :])
    return (logits_mlp, logits_graph)


# speedup vs baseline: 35.2614x; 1.0582x over previous
import functools

import jax
import jax.numpy as jnp
from jax import lax
from jax.experimental import pallas as pl
from jax.experimental.pallas import tpu as pltpu
from jax.experimental.pallas import tpu_sc as plsc

B_SZ = 16
N_INST = 1024
F_IN = 1024
LP = 512
K = 16
N_NODES = 4112
NC, NS = 2, 16
NW = NC * NS


def _pad_cols(a, cols):
    return jnp.pad(a, ((0, 0), (0, cols - a.shape[1])))


def _pad_rows(a, rows):
    return jnp.pad(a, ((0, rows - a.shape[0]), (0, 0)))


def _abmil_body(x_ref, w1_ref, b1_ref, wa1_ref, ba1_ref, wa2t_ref, ba2_ref,
                wc_ref, bc_ref, slide_ref, logits_ref):
    x = x_ref[0]
    h = jnp.maximum(
        jnp.dot(x, w1_ref[...], preferred_element_type=jnp.float32)
        + b1_ref[...], 0.0)
    a = jnp.maximum(
        jnp.dot(h, wa1_ref[...], preferred_element_type=jnp.float32)
        + ba1_ref[...], 0.0)
    s = jnp.sum(a * wa2t_ref[...], axis=1, keepdims=True) + ba2_ref[...]
    m = jnp.max(s, axis=0, keepdims=True)
    e = jnp.exp(s - m)
    w = e / jnp.sum(e, axis=0, keepdims=True)
    slide = jnp.sum(h * w, axis=0, keepdims=True)
    slide_ref[0] = slide
    logits_ref[0] = (
        jnp.dot(slide, wc_ref[...], preferred_element_type=jnp.float32)
        + bc_ref[...])


def _abmil(x, w1p, b1p, wa1p, ba1, wa2t, ba2, wcp, bc):
    return pl.pallas_call(
        _abmil_body,
        grid=(B_SZ,),
        in_specs=[
            pl.BlockSpec((1, N_INST, F_IN), lambda b: (b, 0, 0)),
            pl.BlockSpec((F_IN, LP), lambda b: (0, 0)),
            pl.BlockSpec((1, LP), lambda b: (0, 0)),
            pl.BlockSpec((LP, 128), lambda b: (0, 0)),
            pl.BlockSpec((1, 128), lambda b: (0, 0)),
            pl.BlockSpec((1, 128), lambda b: (0, 0)),
            pl.BlockSpec((1, 1), lambda b: (0, 0)),
            pl.BlockSpec((LP, 2), lambda b: (0, 0)),
            pl.BlockSpec((1, 2), lambda b: (0, 0)),
        ],
        out_specs=[
            pl.BlockSpec((1, 1, LP), lambda b: (b, 0, 0)),
            pl.BlockSpec((1, 1, 2), lambda b: (b, 0, 0)),
        ],
        out_shape=[
            jax.ShapeDtypeStruct((B_SZ, 1, LP), jnp.float32),
            jax.ShapeDtypeStruct((B_SZ, 1, 2), jnp.float32),
        ],
    )(x, w1p, b1p, wa1p, ba1, wa2t, ba2, wcp, bc)


def _embed_body(slide_ref, reh_ref, wg_ref, bg_ref,
                en_ref, xcp_ref, idx_ref, attr_ref):
    s = slide_ref[...]
    r = reh_ref[...]
    e1 = jnp.tanh(
        jnp.dot(s, wg_ref[...], preferred_element_type=jnp.float32)
        + bg_ref[...])
    en1 = e1 / (jnp.sqrt(jnp.sum(e1 * e1, axis=1, keepdims=True)) + 1e-8)
    e2 = jnp.tanh(
        jnp.dot(r, wg_ref[0:500, :], preferred_element_type=jnp.float32)
        + bg_ref[...])
    en2 = e2 / (jnp.sqrt(jnp.sum(e2 * e2, axis=1, keepdims=True)) + 1e-8)
    en_ref[0:B_SZ] = en1
    en_ref[B_SZ:] = en2
    xcp_ref[0:B_SZ] = s
    xcp_ref[B_SZ:] = jnp.concatenate(
        [r, jnp.zeros((r.shape[0], LP - r.shape[1]), jnp.float32)], axis=1)
    sim = jnp.concatenate(
        [lax.dot_general(en1, en1, (((1,), (1,)), ((), ())),
                         preferred_element_type=jnp.float32),
         lax.dot_general(en1, en2, (((1,), (1,)), ((), ())),
                         preferred_element_type=jnp.float32)], axis=1)
    n = sim.shape[1]
    col = lax.broadcasted_iota(jnp.int32, (B_SZ, n), 1)
    vals, idxs = [], []
    cur = sim
    for _ in range(K):
        m = jnp.max(cur, axis=1, keepdims=True)
        am = jnp.min(jnp.where(cur == m, col, n), axis=1, keepdims=True)
        vals.append(m)
        idxs.append(am)
        cur = jnp.where(col == am, -jnp.inf, cur)
    v = jnp.concatenate(vals, axis=1)
    idx_ref[...] = jnp.concatenate(idxs, axis=1)
    ev = jnp.exp(v - v[:, 0:1])
    attr_ref[...] = ev / jnp.sum(ev, axis=1, keepdims=True)


def _embed(slide_p, rehearsal, wgp, bgp):
    return pl.pallas_call(
        _embed_body,
        out_shape=[
            jax.ShapeDtypeStruct((N_NODES, LP), jnp.float32),
            jax.ShapeDtypeStruct((N_NODES, LP), jnp.float32),
            jax.ShapeDtypeStruct((B_SZ, K), jnp.int32),
            jax.ShapeDtypeStruct((B_SZ, K), jnp.float32),
        ],
    )(slide_p, rehearsal, wgp, bgp)


def _topk_body(jidx_ref, en_ref, idx_ref, attr_ref):
    nj = jidx_ref.shape[0]
    n = en_ref.shape[0]
    colg = lax.broadcasted_iota(jnp.int32, (nj, n), 1)
    onehot = jnp.where(colg == jidx_ref[...], 1.0, 0.0)
    rows = jnp.dot(onehot, en_ref[...],
                   preferred_element_type=jnp.float32)
    sim = lax.dot_general(rows, en_ref[...],
                          (((1,), (1,)), ((), ())),
                          preferred_element_type=jnp.float32)
    r, n = sim.shape
    col = lax.broadcasted_iota(jnp.int32, (r, n), 1)
    vals, idxs = [], []
    cur = sim
    for _ in range(K):
        m = jnp.max(cur, axis=1, keepdims=True)
        am = jnp.min(jnp.where(cur == m, col, n), axis=1, keepdims=True)
        vals.append(m)
        idxs.append(am)
        cur = jnp.where(col == am, -jnp.inf, cur)
    v = jnp.concatenate(vals, axis=1)
    idx_ref[...] = jnp.concatenate(idxs, axis=1)
    ev = jnp.exp(v - v[:, 0:1])
    attr_ref[...] = ev / jnp.sum(ev, axis=1, keepdims=True)


def _topk(j_idx, en):
    r = j_idx.shape[0]
    return pl.pallas_call(
        _topk_body,
        out_shape=[
            jax.ShapeDtypeStruct((r, K), jnp.int32),
            jax.ShapeDtypeStruct((r, K), jnp.float32),
        ],
    )(j_idx, en)


def _sc_gather_pool(table, idx, wts):
    b = idx.shape[0]
    d = table.shape[1]
    b_per_w = b // NW
    j_per_w = b_per_w // K
    nch = d // 16
    mesh = plsc.VectorSubcoreMesh(core_axis_name="c", subcore_axis_name="s")

    @functools.partial(
        pl.kernel,
        mesh=mesh,
        out_type=jax.ShapeDtypeStruct((b // K, d), jnp.float32),
        scratch_types=[
            pltpu.VMEM((b_per_w,), jnp.int32),
            pltpu.VMEM((b_per_w,), jnp.float32),
            pltpu.VMEM((b_per_w, d), jnp.float32),
            pltpu.VMEM((j_per_w, d), jnp.float32),
            pltpu.SemaphoreType.DMA,
        ],
    )
    def gather_pool_kernel(table_hbm, idx_hbm, wts_hbm, out_hbm,
                           idx_v, wts_v, rows_v, acc_v, sem):
        wid = lax.axis_index("s") * NC + lax.axis_index("c")
        base = wid * b_per_w
        pltpu.sync_copy(idx_hbm.at[pl.ds(base, b_per_w)], idx_v)
        pltpu.sync_copy(wts_hbm.at[pl.ds(base, b_per_w)], wts_v)
        pltpu.async_copy(table_hbm.at[idx_v], rows_v, sem).wait()

        def pool_row(j, carry):
            w_vec = wts_v[pl.ds(j * K, 16)]
            wks = [lax.gather(
                       w_vec, jnp.full((16, 1), k, jnp.int32),
                       lax.GatherDimensionNumbers(
                           offset_dims=(), collapsed_slice_dims=(0,),
                           start_index_map=(0,)),
                       (1,), mode=lax.GatherScatterMode.PROMISE_IN_BOUNDS)
                   for k in range(K)]
            for c in range(nch):
                acc = jnp.zeros((16,), jnp.float32)
                for k in range(K):
                    acc = acc + wks[k] * rows_v[j * K + k, pl.ds(c * 16, 16)]
                acc_v[j, pl.ds(c * 16, 16)] = acc
            return carry

        lax.fori_loop(0, j_per_w, pool_row, 0)
        pltpu.sync_copy(acc_v, out_hbm.at[pl.ds(wid * j_per_w, j_per_w)])

    return gather_pool_kernel(table, idx, wts)


def _gcn_body(agg1_ref, wgc1_ref, bgc1_ref, attr0_ref, wgc2_ref,
              bgc2_ref, out_ref):
    nj = agg1_ref.shape[0]
    h1 = jnp.maximum(
        jnp.dot(agg1_ref[...], wgc1_ref[...],
                preferred_element_type=jnp.float32)
        + bgc1_ref[...], 0.0)
    row2 = lax.broadcasted_iota(jnp.int32, (B_SZ, nj), 0)
    col2 = lax.broadcasted_iota(jnp.int32, (B_SZ, nj), 1)
    wmat2 = jnp.where(col2 // K == row2, attr0_ref[...], 0.0)
    agg2 = jnp.dot(wmat2, h1, preferred_element_type=jnp.float32)
    out_ref[...] = (
        jnp.dot(agg2, wgc2_ref[...], preferred_element_type=jnp.float32)
        + bgc2_ref[...])


def _gcn(agg1, wgc1p, bgc1p, attr0_flat, wgc2p, bgc2):
    return pl.pallas_call(
        _gcn_body,
        out_shape=jax.ShapeDtypeStruct((B_SZ, 2), jnp.float32),
    )(agg1, wgc1p, bgc1p, attr0_flat, wgc2p, bgc2)


def kernel(x, W1, b1, Wa1, ba1, Wa2, ba2, Wc, bc, rehearsal,
           Wg, bg, Wgc1, bgc1, Wgc2, bgc2):
    f32 = jnp.float32
    w1p = _pad_cols(W1, LP)
    b1p = _pad_cols(b1[None, :], LP)
    wa1p = _pad_rows(Wa1, LP)
    wcp = _pad_rows(Wc, LP)
    wgp = _pad_cols(_pad_rows(Wg, LP), LP)
    bgp = _pad_cols(bg[None, :], LP)
    wgc1p = _pad_cols(_pad_rows(Wgc1, LP), LP)
    bgc1p = _pad_cols(bgc1[None, :], LP)
    wgc2p = _pad_rows(Wgc2, LP)

    slide_p, logits_mlp = _abmil(
        x, w1p, b1p, wa1p, ba1[None, :].astype(f32), Wa2.T.astype(f32),
        ba2[None, :].astype(f32), wcp, bc[None, :].astype(f32))
    slide_p = slide_p[:, 0, :]
    logits_mlp = logits_mlp[:, 0, :]

    en, xcp, idx0, attr0 = _embed(slide_p, rehearsal, wgp, bgp)
    j_idx = idx0.reshape(B_SZ * K, 1)

    idxj, attrj = _topk(j_idx, en)
    src = idxj.reshape(B_SZ * K * K)
    agg1 = _sc_gather_pool(xcp, src, attrj.reshape(B_SZ * K * K))

    logits_graph = _gcn(agg1, wgc1p, bgc1p,
                        attr0.reshape(1, B_SZ * K), wgc2p, bgc2[None, :])
    return (logits_mlp, logits_graph)


# speedup vs baseline: 37.0165x; 1.0498x over previous
import functools

import jax
import jax.numpy as jnp
from jax import lax
from jax.experimental import pallas as pl
from jax.experimental.pallas import tpu as pltpu
from jax.experimental.pallas import tpu_sc as plsc

B_SZ = 16
N_INST = 1024
F_IN = 1024
LP = 512
K = 16
N_NODES = 4112
NC, NS = 2, 16
NW = NC * NS


def _pad_cols(a, cols):
    return jnp.pad(a, ((0, 0), (0, cols - a.shape[1])))


def _pad_rows(a, rows):
    return jnp.pad(a, ((0, rows - a.shape[0]), (0, 0)))


def _abmil_body(x_ref, w1_ref, b1_ref, wa1_ref, ba1_ref, wa2t_ref, ba2_ref,
                wc_ref, bc_ref, slide_ref, logits_ref):
    x = x_ref[0]
    h = jnp.maximum(
        jnp.dot(x, w1_ref[...], preferred_element_type=jnp.float32)
        + b1_ref[...], 0.0)
    a = jnp.maximum(
        jnp.dot(h, wa1_ref[...], preferred_element_type=jnp.float32)
        + ba1_ref[...], 0.0)
    s = jnp.sum(a * wa2t_ref[...], axis=1, keepdims=True) + ba2_ref[...]
    m = jnp.max(s, axis=0, keepdims=True)
    e = jnp.exp(s - m)
    w = e / jnp.sum(e, axis=0, keepdims=True)
    slide = jnp.sum(h * w, axis=0, keepdims=True)
    slide_ref[0] = slide
    logits_ref[0] = (
        jnp.dot(slide, wc_ref[...], preferred_element_type=jnp.float32)
        + bc_ref[...])


def _abmil(x, w1p, b1p, wa1p, ba1, wa2t, ba2, wcp, bc):
    return pl.pallas_call(
        _abmil_body,
        grid=(B_SZ,),
        in_specs=[
            pl.BlockSpec((1, N_INST, F_IN), lambda b: (b, 0, 0)),
            pl.BlockSpec((F_IN, LP), lambda b: (0, 0)),
            pl.BlockSpec((1, LP), lambda b: (0, 0)),
            pl.BlockSpec((LP, 128), lambda b: (0, 0)),
            pl.BlockSpec((1, 128), lambda b: (0, 0)),
            pl.BlockSpec((1, 128), lambda b: (0, 0)),
            pl.BlockSpec((1, 1), lambda b: (0, 0)),
            pl.BlockSpec((LP, 2), lambda b: (0, 0)),
            pl.BlockSpec((1, 2), lambda b: (0, 0)),
        ],
        out_specs=[
            pl.BlockSpec((1, 1, LP), lambda b: (b, 0, 0)),
            pl.BlockSpec((1, 1, 2), lambda b: (b, 0, 0)),
        ],
        out_shape=[
            jax.ShapeDtypeStruct((B_SZ, 1, LP), jnp.float32),
            jax.ShapeDtypeStruct((B_SZ, 1, 2), jnp.float32),
        ],
    )(x, w1p, b1p, wa1p, ba1, wa2t, ba2, wcp, bc)


def _stable_topk(sim):
    r, n = sim.shape
    col = lax.broadcasted_iota(jnp.int32, (r, n), 1)
    vals, idxs = [], []
    cur = sim
    for _ in range(K):
        m = jnp.max(cur, axis=1, keepdims=True)
        am = jnp.min(jnp.where(cur == m, col, n), axis=1, keepdims=True)
        vals.append(m)
        idxs.append(am)
        cur = jnp.where(col == am, -jnp.inf, cur)
    v = jnp.concatenate(vals, axis=1)
    ev = jnp.exp(v - v[:, 0:1])
    attr = ev / jnp.sum(ev, axis=1, keepdims=True)
    return jnp.concatenate(idxs, axis=1), attr


def _graph_body(slide_ref, reh_ref, wg_ref, bg_ref,
                xcp_ref, idxj_ref, attrj_ref, attr0_ref):
    s = slide_ref[...]
    r = reh_ref[...]
    e1 = jnp.tanh(
        jnp.dot(s, wg_ref[...], preferred_element_type=jnp.float32)
        + bg_ref[...])
    en1 = e1 / (jnp.sqrt(jnp.sum(e1 * e1, axis=1, keepdims=True)) + 1e-8)
    e2 = jnp.tanh(
        jnp.dot(r, wg_ref[0:500, :], preferred_element_type=jnp.float32)
        + bg_ref[...])
    en2 = e2 / (jnp.sqrt(jnp.sum(e2 * e2, axis=1, keepdims=True)) + 1e-8)
    en = jnp.concatenate([en1, en2], axis=0)
    xcp_ref[0:B_SZ] = s
    xcp_ref[B_SZ:] = jnp.concatenate(
        [r, jnp.zeros((r.shape[0], LP - r.shape[1]), jnp.float32)], axis=1)
    sim0 = lax.dot_general(en1, en, (((1,), (1,)), ((), ())),
                           preferred_element_type=jnp.float32)
    idx0, attr0 = _stable_topk(sim0)
    attr0_ref[...] = attr0
    nj = B_SZ * K
    rowp = lax.broadcasted_iota(jnp.int32, (nj, K), 0)
    lcol = lax.broadcasted_iota(jnp.int32, (nj, K), 1)
    psel = jnp.where(lcol == rowp // K, 1.0, 0.0)
    x16 = jnp.dot(psel, idx0.astype(jnp.float32),
                  preferred_element_type=jnp.float32)
    jcol = jnp.sum(jnp.where(lcol == rowp % K, x16, 0.0),
                   axis=1, keepdims=True)
    n = en.shape[0]
    colg = lax.broadcasted_iota(jnp.int32, (nj, n), 1).astype(jnp.float32)
    onehot = jnp.where(colg == jcol, 1.0, 0.0)
    rows = jnp.dot(onehot, en, preferred_element_type=jnp.float32)
    sim = lax.dot_general(rows, en, (((1,), (1,)), ((), ())),
                          preferred_element_type=jnp.float32)
    idxj, attrj = _stable_topk(sim)
    idxj_ref[...] = idxj
    attrj_ref[...] = attrj


def _graph(slide_p, rehearsal, wgp, bgp):
    nj = B_SZ * K
    return pl.pallas_call(
        _graph_body,
        out_shape=[
            jax.ShapeDtypeStruct((N_NODES, LP), jnp.float32),
            jax.ShapeDtypeStruct((nj, K), jnp.int32),
            jax.ShapeDtypeStruct((nj, K), jnp.float32),
            jax.ShapeDtypeStruct((B_SZ, K), jnp.float32),
        ],
    )(slide_p, rehearsal, wgp, bgp)


def _sc_gather_pool(table, idx, wts):
    b = idx.shape[0]
    d = table.shape[1]
    b_per_w = b // NW
    j_per_w = b_per_w // K
    nch = d // 16
    mesh = plsc.VectorSubcoreMesh(core_axis_name="c", subcore_axis_name="s")

    @functools.partial(
        pl.kernel,
        mesh=mesh,
        out_type=jax.ShapeDtypeStruct((b // K, d), jnp.float32),
        scratch_types=[
            pltpu.VMEM((b_per_w,), jnp.int32),
            pltpu.VMEM((b_per_w,), jnp.float32),
            pltpu.VMEM((b_per_w, d), jnp.float32),
            pltpu.VMEM((j_per_w, d), jnp.float32),
            pltpu.SemaphoreType.DMA,
        ],
    )
    def gather_pool_kernel(table_hbm, idx_hbm, wts_hbm, out_hbm,
                           idx_v, wts_v, rows_v, acc_v, sem):
        wid = lax.axis_index("s") * NC + lax.axis_index("c")
        base = wid * b_per_w
        pltpu.sync_copy(idx_hbm.at[pl.ds(base, b_per_w)], idx_v)
        pltpu.sync_copy(wts_hbm.at[pl.ds(base, b_per_w)], wts_v)
        pltpu.async_copy(table_hbm.at[idx_v], rows_v, sem).wait()

        def pool_row(j, carry):
            w_vec = wts_v[pl.ds(j * K, 16)]
            wks = [lax.gather(
                       w_vec, jnp.full((16, 1), k, jnp.int32),
                       lax.GatherDimensionNumbers(
                           offset_dims=(), collapsed_slice_dims=(0,),
                           start_index_map=(0,)),
                       (1,), mode=lax.GatherScatterMode.PROMISE_IN_BOUNDS)
                   for k in range(K)]
            for c in range(nch):
                acc = jnp.zeros((16,), jnp.float32)
                for k in range(K):
                    acc = acc + wks[k] * rows_v[j * K + k, pl.ds(c * 16, 16)]
                acc_v[j, pl.ds(c * 16, 16)] = acc
            return carry

        lax.fori_loop(0, j_per_w, pool_row, 0)
        pltpu.sync_copy(acc_v, out_hbm.at[pl.ds(wid * j_per_w, j_per_w)])

    return gather_pool_kernel(table, idx, wts)


def _gcn_body(agg1_ref, wgc1_ref, bgc1_ref, attr0_ref, wgc2_ref,
              bgc2_ref, out_ref):
    nj = agg1_ref.shape[0]
    h1 = jnp.maximum(
        jnp.dot(agg1_ref[...], wgc1_ref[...],
                preferred_element_type=jnp.float32)
        + bgc1_ref[...], 0.0)
    row2 = lax.broadcasted_iota(jnp.int32, (B_SZ, nj), 0)
    col2 = lax.broadcasted_iota(jnp.int32, (B_SZ, nj), 1)
    wmat2 = jnp.where(col2 // K == row2, attr0_ref[...], 0.0)
    agg2 = jnp.dot(wmat2, h1, preferred_element_type=jnp.float32)
    out_ref[...] = (
        jnp.dot(agg2, wgc2_ref[...], preferred_element_type=jnp.float32)
        + bgc2_ref[...])


def _gcn(agg1, wgc1p, bgc1p, attr0_flat, wgc2p, bgc2):
    return pl.pallas_call(
        _gcn_body,
        out_shape=jax.ShapeDtypeStruct((B_SZ, 2), jnp.float32),
    )(agg1, wgc1p, bgc1p, attr0_flat, wgc2p, bgc2)


def kernel(x, W1, b1, Wa1, ba1, Wa2, ba2, Wc, bc, rehearsal,
           Wg, bg, Wgc1, bgc1, Wgc2, bgc2):
    f32 = jnp.float32
    w1p = _pad_cols(W1, LP)
    b1p = _pad_cols(b1[None, :], LP)
    wa1p = _pad_rows(Wa1, LP)
    wcp = _pad_rows(Wc, LP)
    wgp = _pad_cols(_pad_rows(Wg, LP), LP)
    bgp = _pad_cols(bg[None, :], LP)
    wgc1p = _pad_cols(_pad_rows(Wgc1, LP), LP)
    bgc1p = _pad_cols(bgc1[None, :], LP)
    wgc2p = _pad_rows(Wgc2, LP)

    slide_p, logits_mlp = _abmil(
        x, w1p, b1p, wa1p, ba1[None, :].astype(f32), Wa2.T.astype(f32),
        ba2[None, :].astype(f32), wcp, bc[None, :].astype(f32))
    slide_p = slide_p[:, 0, :]
    logits_mlp = logits_mlp[:, 0, :]

    xcp, idxj, attrj, attr0 = _graph(slide_p, rehearsal, wgp, bgp)
    src = idxj.reshape(B_SZ * K * K)
    agg1 = _sc_gather_pool(xcp, src, attrj.reshape(B_SZ * K * K))

    logits_graph = _gcn(agg1, wgc1p, bgc1p,
                        attr0.reshape(1, B_SZ * K), wgc2p, bgc2[None, :])
    return (logits_mlp, logits_graph)


# speedup vs baseline: 37.6430x; 1.0169x over previous
import functools

import jax
import jax.numpy as jnp
from jax import lax
from jax.experimental import pallas as pl
from jax.experimental.pallas import tpu as pltpu
from jax.experimental.pallas import tpu_sc as plsc

B_SZ = 16
N_INST = 1024
F_IN = 1024
LP = 512
K = 16
N_NODES = 4112
NC, NS = 2, 16
NW = NC * NS


def _pad_cols(a, cols):
    return jnp.pad(a, ((0, 0), (0, cols - a.shape[1])))


def _pad_rows(a, rows):
    return jnp.pad(a, ((0, rows - a.shape[0]), (0, 0)))


def _stable_topk(sim):
    r, n = sim.shape
    col = lax.broadcasted_iota(jnp.int32, (r, n), 1)
    vals, idxs = [], []
    cur = sim
    for _ in range(K):
        m = jnp.max(cur, axis=1, keepdims=True)
        am = jnp.min(jnp.where(cur == m, col, n), axis=1, keepdims=True)
        vals.append(m)
        idxs.append(am)
        cur = jnp.where(col == am, -jnp.inf, cur)
    v = jnp.concatenate(vals, axis=1)
    ev = jnp.exp(v - v[:, 0:1])
    attr = ev / jnp.sum(ev, axis=1, keepdims=True)
    return jnp.concatenate(idxs, axis=1), attr


def _graph_body(x_ref, w1_ref, b1_ref, wa1_ref, ba1_ref, wa2t_ref, ba2_ref,
                wc_ref, bc_ref, reh_ref, wg_ref, bg_ref,
                logits_ref, xcp_ref, idxj_ref, attrj_ref, attr0_ref,
                slide_scr):
    b = pl.program_id(0)

    @pl.when(b < B_SZ)
    def _abmil_step():
        x = x_ref[0]
        h = jnp.maximum(
            jnp.dot(x, w1_ref[...], preferred_element_type=jnp.float32)
            + b1_ref[...], 0.0)
        a = jnp.maximum(
            jnp.dot(h, wa1_ref[...], preferred_element_type=jnp.float32)
            + ba1_ref[...], 0.0)
        sc = jnp.sum(a * wa2t_ref[...], axis=1, keepdims=True) + ba2_ref[...]
        m = jnp.max(sc, axis=0, keepdims=True)
        e = jnp.exp(sc - m)
        w = e / jnp.sum(e, axis=0, keepdims=True)
        slide = jnp.sum(h * w, axis=0, keepdims=True)
        slide_scr[pl.ds(b, 1), :] = slide
        logits_ref[0] = (
            jnp.dot(slide, wc_ref[...], preferred_element_type=jnp.float32)
            + bc_ref[...])

    @pl.when(b == B_SZ)
    def _graph_step():
        _graph_tail(slide_scr, reh_ref, wg_ref, bg_ref,
                    xcp_ref, idxj_ref, attrj_ref, attr0_ref)


def _graph_tail(slide_ref, reh_ref, wg_ref, bg_ref,
                xcp_ref, idxj_ref, attrj_ref, attr0_ref):
    s = slide_ref[...]
    r = reh_ref[...]
    e1 = jnp.tanh(
        jnp.dot(s, wg_ref[...], preferred_element_type=jnp.float32)
        + bg_ref[...])
    en1 = e1 / (jnp.sqrt(jnp.sum(e1 * e1, axis=1, keepdims=True)) + 1e-8)
    e2 = jnp.tanh(
        jnp.dot(r, wg_ref[0:500, :], preferred_element_type=jnp.float32)
        + bg_ref[...])
    en2 = e2 / (jnp.sqrt(jnp.sum(e2 * e2, axis=1, keepdims=True)) + 1e-8)
    en = jnp.concatenate([en1, en2], axis=0)
    xcp_ref[0:B_SZ] = s
    xcp_ref[B_SZ:] = jnp.concatenate(
        [r, jnp.zeros((r.shape[0], LP - r.shape[1]), jnp.float32)], axis=1)
    sim0 = lax.dot_general(en1, en, (((1,), (1,)), ((), ())),
                           preferred_element_type=jnp.float32)
    idx0, attr0 = _stable_topk(sim0)
    attr0_ref[...] = attr0
    nj = B_SZ * K
    rowp = lax.broadcasted_iota(jnp.int32, (nj, K), 0)
    lcol = lax.broadcasted_iota(jnp.int32, (nj, K), 1)
    psel = jnp.where(lcol == rowp // K, 1.0, 0.0)
    x16 = jnp.dot(psel, idx0.astype(jnp.float32),
                  preferred_element_type=jnp.float32)
    jcol = jnp.sum(jnp.where(lcol == rowp % K, x16, 0.0),
                   axis=1, keepdims=True)
    n = en.shape[0]
    colg = lax.broadcasted_iota(jnp.int32, (nj, n), 1).astype(jnp.float32)
    onehot = jnp.where(colg == jcol, 1.0, 0.0)
    rows = jnp.dot(onehot, en, preferred_element_type=jnp.float32)
    sim = lax.dot_general(rows, en, (((1,), (1,)), ((), ())),
                          preferred_element_type=jnp.float32)
    idxj, attrj = _stable_topk(sim)
    idxj_ref[...] = idxj
    attrj_ref[...] = attrj


def _graph(x, w1p, b1p, wa1p, ba1, wa2t, ba2, wcp, bc, rehearsal, wgp, bgp):
    nj = B_SZ * K
    const = lambda b: (0, 0)
    return pl.pallas_call(
        _graph_body,
        grid=(B_SZ + 1,),
        in_specs=[
            pl.BlockSpec((1, N_INST, F_IN),
                         lambda b: (jnp.minimum(b, B_SZ - 1), 0, 0)),
            pl.BlockSpec((F_IN, LP), const),
            pl.BlockSpec((1, LP), const),
            pl.BlockSpec((LP, 128), const),
            pl.BlockSpec((1, 128), const),
            pl.BlockSpec((1, 128), const),
            pl.BlockSpec((1, 1), const),
            pl.BlockSpec((LP, 2), const),
            pl.BlockSpec((1, 2), const),
            pl.BlockSpec((4096, 500), const),
            pl.BlockSpec((LP, LP), const),
            pl.BlockSpec((1, LP), const),
        ],
        out_specs=[
            pl.BlockSpec((1, 1, 2), lambda b: (jnp.minimum(b, B_SZ - 1), 0, 0)),
            pl.BlockSpec((N_NODES, LP), const),
            pl.BlockSpec((nj, K), const),
            pl.BlockSpec((nj, K), const),
            pl.BlockSpec((B_SZ, K), const),
        ],
        out_shape=[
            jax.ShapeDtypeStruct((B_SZ, 1, 2), jnp.float32),
            jax.ShapeDtypeStruct((N_NODES, LP), jnp.float32),
            jax.ShapeDtypeStruct((nj, K), jnp.int32),
            jax.ShapeDtypeStruct((nj, K), jnp.float32),
            jax.ShapeDtypeStruct((B_SZ, K), jnp.float32),
        ],
        scratch_shapes=[pltpu.VMEM((B_SZ, LP), jnp.float32)],
    )(x, w1p, b1p, wa1p, ba1, wa2t, ba2, wcp, bc, rehearsal, wgp, bgp)


def _sc_gather_pool(table, idx, wts):
    b = idx.shape[0]
    d = table.shape[1]
    b_per_w = b // NW
    j_per_w = b_per_w // K
    nch = d // 16
    mesh = plsc.VectorSubcoreMesh(core_axis_name="c", subcore_axis_name="s")

    @functools.partial(
        pl.kernel,
        mesh=mesh,
        out_type=jax.ShapeDtypeStruct((b // K, d), jnp.float32),
        scratch_types=[
            pltpu.VMEM((b_per_w,), jnp.int32),
            pltpu.VMEM((b_per_w,), jnp.float32),
            pltpu.VMEM((b_per_w, d), jnp.float32),
            pltpu.VMEM((j_per_w, d), jnp.float32),
            pltpu.SemaphoreType.DMA,
        ],
    )
    def gather_pool_kernel(table_hbm, idx_hbm, wts_hbm, out_hbm,
                           idx_v, wts_v, rows_v, acc_v, sem):
        wid = lax.axis_index("s") * NC + lax.axis_index("c")
        base = wid * b_per_w
        pltpu.sync_copy(idx_hbm.at[pl.ds(base, b_per_w)], idx_v)
        pltpu.sync_copy(wts_hbm.at[pl.ds(base, b_per_w)], wts_v)
        pltpu.async_copy(table_hbm.at[idx_v], rows_v, sem).wait()

        def pool_row(j, carry):
            w_vec = wts_v[pl.ds(j * K, 16)]
            wks = [lax.gather(
                       w_vec, jnp.full((16, 1), k, jnp.int32),
                       lax.GatherDimensionNumbers(
                           offset_dims=(), collapsed_slice_dims=(0,),
                           start_index_map=(0,)),
                       (1,), mode=lax.GatherScatterMode.PROMISE_IN_BOUNDS)
                   for k in range(K)]
            for c in range(nch):
                acc = jnp.zeros((16,), jnp.float32)
                for k in range(K):
                    acc = acc + wks[k] * rows_v[j * K + k, pl.ds(c * 16, 16)]
                acc_v[j, pl.ds(c * 16, 16)] = acc
            return carry

        lax.fori_loop(0, j_per_w, pool_row, 0)
        pltpu.sync_copy(acc_v, out_hbm.at[pl.ds(wid * j_per_w, j_per_w)])

    return gather_pool_kernel(table, idx, wts)


def _gcn_body(agg1_ref, wgc1_ref, bgc1_ref, attr0_ref, wgc2_ref,
              bgc2_ref, out_ref):
    nj = agg1_ref.shape[0]
    h1 = jnp.maximum(
        jnp.dot(agg1_ref[...], wgc1_ref[...],
                preferred_element_type=jnp.float32)
        + bgc1_ref[...], 0.0)
    row2 = lax.broadcasted_iota(jnp.int32, (B_SZ, nj), 0)
    col2 = lax.broadcasted_iota(jnp.int32, (B_SZ, nj), 1)
    wmat2 = jnp.where(col2 // K == row2, attr0_ref[...], 0.0)
    agg2 = jnp.dot(wmat2, h1, preferred_element_type=jnp.float32)
    out_ref[...] = (
        jnp.dot(agg2, wgc2_ref[...], preferred_element_type=jnp.float32)
        + bgc2_ref[...])


def _gcn(agg1, wgc1p, bgc1p, attr0_flat, wgc2p, bgc2):
    return pl.pallas_call(
        _gcn_body,
        out_shape=jax.ShapeDtypeStruct((B_SZ, 2), jnp.float32),
    )(agg1, wgc1p, bgc1p, attr0_flat, wgc2p, bgc2)


def kernel(x, W1, b1, Wa1, ba1, Wa2, ba2, Wc, bc, rehearsal,
           Wg, bg, Wgc1, bgc1, Wgc2, bgc2):
    f32 = jnp.float32
    w1p = _pad_cols(W1, LP)
    b1p = _pad_cols(b1[None, :], LP)
    wa1p = _pad_rows(Wa1, LP)
    wcp = _pad_rows(Wc, LP)
    wgp = _pad_cols(_pad_rows(Wg, LP), LP)
    bgp = _pad_cols(bg[None, :], LP)
    wgc1p = _pad_cols(_pad_rows(Wgc1, LP), LP)
    bgc1p = _pad_cols(bgc1[None, :], LP)
    wgc2p = _pad_rows(Wgc2, LP)

    logits_mlp, xcp, idxj, attrj, attr0 = _graph(
        x, w1p, b1p, wa1p, ba1[None, :].astype(f32), Wa2.T.astype(f32),
        ba2[None, :].astype(f32), wcp, bc[None, :].astype(f32),
        rehearsal, wgp, bgp)
    logits_mlp = logits_mlp[:, 0, :]
    src = idxj.reshape(B_SZ * K * K)
    agg1 = _sc_gather_pool(xcp, src, attrj.reshape(B_SZ * K * K))

    logits_graph = _gcn(agg1, wgc1p, bgc1p,
                        attr0.reshape(1, B_SZ * K), wgc2p, bgc2[None, :])
    return (logits_mlp, logits_graph)
